# Initial kernel scaffold; baseline (speedup 1.0000x reference)
import jax
import jax.numpy as jnp
from jax.experimental import pallas as pl


def kernel(q, k, v):
    raise NotImplementedError("write your pallas kernel here")


# speedup vs baseline: 1.1788x; 1.1788x over previous
import jax
import jax.numpy as jnp
from jax.experimental import pallas as pl
from jax.experimental.pallas import tpu as pltpu

_BUCKET = 128


def _sinkhorn_attn_kernel(q_ref, k_ref, v_ref, o_ref):
    t, dh = q_ref.shape[1], q_ref.shape[2]
    nb = t // _BUCKET
    scale = dh ** -0.5

    sq = jnp.concatenate(
        [jnp.mean(q_ref[0, u * _BUCKET:(u + 1) * _BUCKET, :], axis=0,
                  keepdims=True) for u in range(nb)], axis=0)
    sk = jnp.concatenate(
        [jnp.mean(k_ref[0, u * _BUCKET:(u + 1) * _BUCKET, :], axis=0,
                  keepdims=True) for u in range(nb)], axis=0)
    r = jax.lax.dot_general(sq, sk, (((1,), (1,)), ((), ())),
                            preferred_element_type=jnp.float32) * scale

    iota_row = jax.lax.broadcasted_iota(jnp.int32, (1, nb), 1)

    for u in range(nb):
        row = jax.lax.slice(r, (u, 0), (u + 1, nb))
        m = jnp.max(row)
        w_u = 1.0 / jnp.sum(jnp.exp(row - m))
        idx_u = jnp.min(jnp.where(row >= m, iota_row, nb))

        qb = q_ref[0, u * _BUCKET:(u + 1) * _BUCKET, :]
        kl = k_ref[0, u * _BUCKET:(u + 1) * _BUCKET, :]
        vl = v_ref[0, u * _BUCKET:(u + 1) * _BUCKET, :]
        kg = k_ref[0, pl.ds(idx_u * _BUCKET, _BUCKET), :]
        vg = v_ref[0, pl.ds(idx_u * _BUCKET, _BUCKET), :]

        kcat = jnp.concatenate([kg * w_u, kl], axis=0)
        vcat = jnp.concatenate([vg * w_u, vl], axis=0)
        s = jax.lax.dot_general(qb, kcat, (((1,), (1,)), ((), ())),
                                preferred_element_type=jnp.float32) * scale
        smax = jnp.max(s, axis=1, keepdims=True)
        p = jnp.exp(s - smax)
        p = p / jnp.sum(p, axis=1, keepdims=True)
        o = jax.lax.dot_general(p, vcat, (((1,), (0,)), ((), ())),
                                preferred_element_type=jnp.float32)
        o_ref[0, u * _BUCKET:(u + 1) * _BUCKET, :] = o


def kernel(q, k, v):
    b, h, t, dh = q.shape
    bh = b * h
    qm = q.reshape(bh, t, dh)
    km = k.reshape(bh, t, dh)
    vm = v.reshape(bh, t, dh)
    out = pl.pallas_call(
        _sinkhorn_attn_kernel,
        grid=(bh,),
        in_specs=[
            pl.BlockSpec((1, t, dh), lambda i: (i, 0, 0)),
            pl.BlockSpec((1, t, dh), lambda i: (i, 0, 0)),
            pl.BlockSpec((1, t, dh), lambda i: (i, 0, 0)),
        ],
        out_specs=pl.BlockSpec((1, t, dh), lambda i: (i, 0, 0)),
        out_shape=jax.ShapeDtypeStruct((bh, t, dh), q.dtype),
        compiler_params=pltpu.CompilerParams(
            dimension_semantics=("arbitrary",)),
    )(qm, km, vm)
    return out.reshape(b, h, t, dh)
